# Initial kernel scaffold; baseline (speedup 1.0000x reference)
import jax
import jax.numpy as jnp
from jax.experimental import pallas as pl


def kernel(x, edge_index, W1, b1, gamma1, beta1, W2, b2):
    raise NotImplementedError("write your pallas kernel here")


# speedup vs baseline: 8.1239x; 8.1239x over previous
import functools

import jax
import jax.numpy as jnp
from jax import lax
from jax.experimental import pallas as pl
from jax.experimental.pallas import tpu as pltpu
from jax.experimental.pallas import tpu_sc as plsc

NN = 10000
EE = 320000
D = 128
NC = 2
NS = 16
NW = NC * NS
K = 128
CH = 80
EP = NW * CH * K
NP = 10112
RPT = NP // NS

_MESH = plsc.VectorSubcoreMesh(core_axis_name="c", subcore_axis_name="s",
                               num_cores=NC, num_subcores=NS)


def _deg_body(dst_hbm, z_hbm, onew_hbm, deg_out, dst_v, one_v, degS, sem):
    c = lax.axis_index("c")
    s = lax.axis_index("s")
    wid = c * NS + s
    pltpu.sync_copy(dst_hbm.at[wid], dst_v)
    pltpu.sync_copy(onew_hbm, one_v)
    pltpu.sync_copy(z_hbm.at[pl.ds(s * RPT, RPT)], degS.at[pl.ds(s * RPT, RPT)])
    plsc.subcore_barrier()

    def body(j, carry):
        pltpu.sync_copy(one_v, degS.at[dst_v.at[j]], add=True)
        return carry

    lax.fori_loop(0, CH, body, 0)
    plsc.subcore_barrier()
    pltpu.sync_copy(degS.at[pl.ds(s * RPT, RPT)],
                    deg_out.at[c, pl.ds(s * RPT, RPT)])


_deg_call = pl.kernel(
    _deg_body,
    out_type=jax.ShapeDtypeStruct((NC, NP, D), jnp.float32),
    mesh=_MESH,
    scratch_types=[
        pltpu.VMEM((CH, K), jnp.int32),
        pltpu.VMEM((K, D), jnp.float32),
        pltpu.VMEM_SHARED((NP, D), jnp.float32),
        pltpu.SemaphoreType.DMA,
    ],
)


def _agg_body(g_hbm, src_hbm, dst_hbm, z_hbm, s_out, src_v, dst_v, rows, S, sem):
    c = lax.axis_index("c")
    s = lax.axis_index("s")
    wid = c * NS + s
    pltpu.sync_copy(src_hbm.at[wid], src_v)
    pltpu.sync_copy(dst_hbm.at[wid], dst_v)
    pltpu.sync_copy(z_hbm.at[pl.ds(s * RPT, RPT)], S.at[pl.ds(s * RPT, RPT)])
    plsc.subcore_barrier()

    def body(j, carry):
        pltpu.async_copy(g_hbm.at[src_v.at[j]], rows, sem).wait()
        pltpu.sync_copy(rows, S.at[dst_v.at[j]], add=True)
        return carry

    lax.fori_loop(0, CH, body, 0)
    plsc.subcore_barrier()
    pltpu.sync_copy(S.at[pl.ds(s * RPT, RPT)],
                    s_out.at[c, pl.ds(s * RPT, RPT)])


_agg_call = pl.kernel(
    _agg_body,
    out_type=jax.ShapeDtypeStruct((NC, NP, D), jnp.float32),
    mesh=_MESH,
    scratch_types=[
        pltpu.VMEM((CH, K), jnp.int32),
        pltpu.VMEM((CH, K), jnp.int32),
        pltpu.VMEM((K, D), jnp.float32),
        pltpu.VMEM_SHARED((NP, D), jnp.float32),
        pltpu.SemaphoreType.DMA,
    ],
)


def _dinv_of(deg_ref):
    return lax.rsqrt(deg_ref[0, :NN, 0:1] + deg_ref[1, :NN, 0:1] + 1.0)


def _tc1_body(x_ref, w1_ref, deg_ref, g1_ref):
    dinv = _dinv_of(deg_ref)
    h = jnp.dot(x_ref[...], w1_ref[...], preferred_element_type=jnp.float32)
    g1_ref[...] = h * dinv


_tc1_call = pl.pallas_call(
    _tc1_body,
    out_shape=jax.ShapeDtypeStruct((NN, D), jnp.float32),
)


def _tc2_body(s_ref, g1_ref, deg_ref, b1_ref, ga_ref, be_ref, w2_ref, g2_ref):
    dinv = _dinv_of(deg_ref)
    a = dinv * (s_ref[0, :NN, :] + s_ref[1, :NN, :] + g1_ref[...]) + b1_ref[...]
    mu = jnp.mean(a, axis=0, keepdims=True)
    var = jnp.mean((a - mu) ** 2, axis=0, keepdims=True)
    h = (a - mu) * lax.rsqrt(var + 1e-5) * ga_ref[...] + be_ref[...]
    h = jnp.maximum(h, 0.0)
    h2 = jnp.dot(h, w2_ref[...], preferred_element_type=jnp.float32)
    g2_ref[...] = h2 * dinv


_tc2_call = pl.pallas_call(
    _tc2_body,
    out_shape=jax.ShapeDtypeStruct((NN, D), jnp.float32),
)


def _tc3_body(s_ref, g2_ref, deg_ref, b2_ref, o_ref):
    dinv = _dinv_of(deg_ref)
    o_ref[...] = dinv * (s_ref[0, :NN, :] + s_ref[1, :NN, :] + g2_ref[...]) \
        + b2_ref[...]


_tc3_call = pl.pallas_call(
    _tc3_body,
    out_shape=jax.ShapeDtypeStruct((NN, D), jnp.float32),
)


def kernel(x, edge_index, W1, b1, gamma1, beta1, W2, b2):
    x2 = x.reshape(NN, D)
    src = edge_index[0]
    dst = edge_index[1]
    pad = EP - EE
    srcp = jnp.concatenate([src, jnp.zeros((pad,), jnp.int32)]).reshape(NW, CH, K)
    dstp = jnp.concatenate([dst, jnp.full((pad,), NN, jnp.int32)]).reshape(NW, CH, K)
    z128 = jnp.zeros((NP, D), jnp.float32)
    onew = jnp.ones((K, D), jnp.float32)

    degA = _deg_call(dstp, z128, onew)
    g1 = _tc1_call(x2, W1, degA)
    S1 = _agg_call(g1, srcp, dstp, z128)
    g2 = _tc2_call(S1, g1, degA, b1.reshape(1, D), gamma1.reshape(1, D),
                   beta1.reshape(1, D), W2)
    S2 = _agg_call(g2, srcp, dstp, z128)
    out = _tc3_call(S2, g2, degA, b2.reshape(1, D))
    return out.reshape(1, NN, D)


# speedup vs baseline: 9.0325x; 1.1118x over previous
import functools

import jax
import jax.numpy as jnp
from jax import lax
from jax.experimental import pallas as pl
from jax.experimental.pallas import tpu as pltpu
from jax.experimental.pallas import tpu_sc as plsc

NN = 10000
EE = 320000
D = 128
NC = 2
NS = 16
NW = NC * NS
K = 128
CH = 80
EP = NW * CH * K
NP = 10112
RPT = NP // NS

_MESH = plsc.VectorSubcoreMesh(core_axis_name="c", subcore_axis_name="s",
                               num_cores=NC, num_subcores=NS)


def _deg_body(dst_hbm, z_hbm, onew_hbm, deg_out, dst_v, one_v, degS, sem):
    c = lax.axis_index("c")
    s = lax.axis_index("s")
    wid = c * NS + s
    pltpu.sync_copy(dst_hbm.at[wid], dst_v)
    pltpu.sync_copy(onew_hbm, one_v)
    pltpu.sync_copy(z_hbm.at[pl.ds(s * RPT, RPT)], degS.at[pl.ds(s * RPT, RPT)])
    plsc.subcore_barrier()

    def body(j, carry):
        pltpu.sync_copy(one_v, degS.at[dst_v.at[j]], add=True)
        return carry

    lax.fori_loop(0, CH, body, 0)
    plsc.subcore_barrier()
    pltpu.sync_copy(degS.at[pl.ds(s * RPT, RPT)],
                    deg_out.at[c, pl.ds(s * RPT, RPT)])


_deg_call = pl.kernel(
    _deg_body,
    out_type=jax.ShapeDtypeStruct((NC, NP, D), jnp.float32),
    mesh=_MESH,
    scratch_types=[
        pltpu.VMEM((CH, K), jnp.int32),
        pltpu.VMEM((K, D), jnp.float32),
        pltpu.VMEM_SHARED((NP, D), jnp.float32),
        pltpu.SemaphoreType.DMA,
    ],
)


CH2 = CH // 2


def _agg_body(g_hbm, src_hbm, dst_hbm, z_hbm, s_out, src_v, dst_v, rows0,
              rows1, S, sem0, sem1):
    c = lax.axis_index("c")
    s = lax.axis_index("s")
    wid = c * NS + s
    pltpu.sync_copy(z_hbm.at[pl.ds(s * RPT, RPT)], S.at[pl.ds(s * RPT, RPT)])
    plsc.subcore_barrier()

    for half in range(2):
        pltpu.sync_copy(src_hbm.at[wid, pl.ds(half * CH2, CH2)], src_v)
        pltpu.sync_copy(dst_hbm.at[wid, pl.ds(half * CH2, CH2)], dst_v)
        pltpu.async_copy(g_hbm.at[src_v.at[0]], rows0, sem0)
        pltpu.async_copy(g_hbm.at[src_v.at[1]], rows1, sem1)

        def body(t, carry):
            j = 2 * t
            pltpu.make_async_copy(g_hbm.at[src_v.at[j]], rows0, sem0).wait()
            pltpu.sync_copy(rows0, S.at[dst_v.at[j]], add=True)

            @pl.when(j + 2 < CH2)
            def _():
                pltpu.async_copy(g_hbm.at[src_v.at[j + 2]], rows0, sem0)

            pltpu.make_async_copy(g_hbm.at[src_v.at[j + 1]], rows1,
                                  sem1).wait()
            pltpu.sync_copy(rows1, S.at[dst_v.at[j + 1]], add=True)

            @pl.when(j + 3 < CH2)
            def _():
                pltpu.async_copy(g_hbm.at[src_v.at[j + 3]], rows1, sem1)

            return carry

        lax.fori_loop(0, CH2 // 2, body, 0)

    plsc.subcore_barrier()
    pltpu.sync_copy(S.at[pl.ds(s * RPT, RPT)],
                    s_out.at[c, pl.ds(s * RPT, RPT)])


_agg_call = pl.kernel(
    _agg_body,
    out_type=jax.ShapeDtypeStruct((NC, NP, D), jnp.float32),
    mesh=_MESH,
    scratch_types=[
        pltpu.VMEM((CH2, K), jnp.int32),
        pltpu.VMEM((CH2, K), jnp.int32),
        pltpu.VMEM((K, D), jnp.float32),
        pltpu.VMEM((K, D), jnp.float32),
        pltpu.VMEM_SHARED((NP, D), jnp.float32),
        pltpu.SemaphoreType.DMA,
        pltpu.SemaphoreType.DMA,
    ],
)


def _dinv_of(deg_ref):
    return lax.rsqrt(deg_ref[0, :NN, 0:1] + deg_ref[1, :NN, 0:1] + 1.0)


def _tc1_body(x_ref, w1_ref, deg_ref, g1_ref):
    dinv = _dinv_of(deg_ref)
    h = jnp.dot(x_ref[...], w1_ref[...], preferred_element_type=jnp.float32)
    g1_ref[...] = h * dinv


_tc1_call = pl.pallas_call(
    _tc1_body,
    out_shape=jax.ShapeDtypeStruct((NN, D), jnp.float32),
)


def _tc2_body(s_ref, g1_ref, deg_ref, b1_ref, ga_ref, be_ref, w2_ref, g2_ref):
    dinv = _dinv_of(deg_ref)
    a = dinv * (s_ref[0, :NN, :] + s_ref[1, :NN, :] + g1_ref[...]) + b1_ref[...]
    mu = jnp.mean(a, axis=0, keepdims=True)
    var = jnp.mean((a - mu) ** 2, axis=0, keepdims=True)
    h = (a - mu) * lax.rsqrt(var + 1e-5) * ga_ref[...] + be_ref[...]
    h = jnp.maximum(h, 0.0)
    h2 = jnp.dot(h, w2_ref[...], preferred_element_type=jnp.float32)
    g2_ref[...] = h2 * dinv


_tc2_call = pl.pallas_call(
    _tc2_body,
    out_shape=jax.ShapeDtypeStruct((NN, D), jnp.float32),
)


def _tc3_body(s_ref, g2_ref, deg_ref, b2_ref, o_ref):
    dinv = _dinv_of(deg_ref)
    o_ref[...] = dinv * (s_ref[0, :NN, :] + s_ref[1, :NN, :] + g2_ref[...]) \
        + b2_ref[...]


_tc3_call = pl.pallas_call(
    _tc3_body,
    out_shape=jax.ShapeDtypeStruct((NN, D), jnp.float32),
)


def kernel(x, edge_index, W1, b1, gamma1, beta1, W2, b2):
    x2 = x.reshape(NN, D)
    src = edge_index[0]
    dst = edge_index[1]
    pad = EP - EE
    srcp = jnp.concatenate([src, jnp.zeros((pad,), jnp.int32)]).reshape(NW, CH, K)
    dstp = jnp.concatenate([dst, jnp.full((pad,), NN, jnp.int32)]).reshape(NW, CH, K)
    z128 = jnp.zeros((NP, D), jnp.float32)
    onew = jnp.ones((K, D), jnp.float32)

    degA = _deg_call(dstp, z128, onew)
    g1 = _tc1_call(x2, W1, degA)
    S1 = _agg_call(g1, srcp, dstp, z128)
    g2 = _tc2_call(S1, g1, degA, b1.reshape(1, D), gamma1.reshape(1, D),
                   beta1.reshape(1, D), W2)
    S2 = _agg_call(g2, srcp, dstp, z128)
    out = _tc3_call(S2, g2, degA, b2.reshape(1, D))
    return out.reshape(1, NN, D)


# speedup vs baseline: 9.5746x; 1.0600x over previous
import functools

import jax
import jax.numpy as jnp
from jax import lax
from jax.experimental import pallas as pl
from jax.experimental.pallas import tpu as pltpu
from jax.experimental.pallas import tpu_sc as plsc

NN = 10000
EE = 320000
D = 128
NC = 2
NS = 16
NW = NC * NS
K = 128
CH = 80
EP = NW * CH * K
NP = 10112
RPT = NP // NS

_MESH = plsc.VectorSubcoreMesh(core_axis_name="c", subcore_axis_name="s",
                               num_cores=NC, num_subcores=NS)


def _deg_body(dst_hbm, z_hbm, onew_hbm, deg_out, dst_v, one_v, degS, sem):
    c = lax.axis_index("c")
    s = lax.axis_index("s")
    wid = c * NS + s
    pltpu.sync_copy(dst_hbm.at[wid], dst_v)
    pltpu.sync_copy(onew_hbm, one_v)
    pltpu.sync_copy(z_hbm.at[pl.ds(s * RPT, RPT)], degS.at[pl.ds(s * RPT, RPT)])
    plsc.subcore_barrier()

    def body(j, carry):
        pltpu.sync_copy(one_v, degS.at[dst_v.at[j]], add=True)
        return carry

    lax.fori_loop(0, CH, body, 0)
    plsc.subcore_barrier()
    pltpu.sync_copy(degS.at[pl.ds(s * RPT, RPT)],
                    deg_out.at[c, pl.ds(s * RPT, RPT)])


_deg_call = pl.kernel(
    _deg_body,
    out_type=jax.ShapeDtypeStruct((NC, NP, D), jnp.float32),
    mesh=_MESH,
    scratch_types=[
        pltpu.VMEM((CH, K), jnp.int32),
        pltpu.VMEM((K, D), jnp.float32),
        pltpu.VMEM_SHARED((NP, D), jnp.float32),
        pltpu.SemaphoreType.DMA,
    ],
)


KA = 64
NBUF = 4
CHA = EP // (NW * KA)
NQ = 4
CHQ = CHA // NQ


def _agg_body(g_hbm, src_hbm, dst_hbm, z_hbm, s_out, src_v, dst_v, rows0,
              rows1, rows2, rows3, S, sem0, sem1, sem2, sem3):
    c = lax.axis_index("c")
    s = lax.axis_index("s")
    wid = c * NS + s
    rows = (rows0, rows1, rows2, rows3)
    sems = (sem0, sem1, sem2, sem3)
    pltpu.sync_copy(z_hbm.at[pl.ds(s * RPT, RPT)], S.at[pl.ds(s * RPT, RPT)])
    plsc.subcore_barrier()

    for q in range(NQ):
        pltpu.sync_copy(src_hbm.at[wid, pl.ds(q * CHQ, CHQ)], src_v)
        pltpu.sync_copy(dst_hbm.at[wid, pl.ds(q * CHQ, CHQ)], dst_v)
        for b in range(NBUF):
            pltpu.async_copy(g_hbm.at[src_v.at[b]], rows[b], sems[b])

        def body(t, carry):
            j0 = NBUF * t
            for b in range(NBUF):
                j = j0 + b
                pltpu.make_async_copy(g_hbm.at[src_v.at[j]], rows[b],
                                      sems[b]).wait()
                pltpu.sync_copy(rows[b], S.at[dst_v.at[j]], add=True)

                @pl.when(j + NBUF < CHQ)
                def _():
                    pltpu.async_copy(g_hbm.at[src_v.at[j + NBUF]], rows[b],
                                     sems[b])

            return carry

        lax.fori_loop(0, CHQ // NBUF, body, 0)

    plsc.subcore_barrier()
    pltpu.sync_copy(S.at[pl.ds(s * RPT, RPT)],
                    s_out.at[c, pl.ds(s * RPT, RPT)])


_agg_call = pl.kernel(
    _agg_body,
    out_type=jax.ShapeDtypeStruct((NC, NP, D), jnp.float32),
    mesh=_MESH,
    scratch_types=[
        pltpu.VMEM((CHQ, KA), jnp.int32),
        pltpu.VMEM((CHQ, KA), jnp.int32),
        pltpu.VMEM((KA, D), jnp.float32),
        pltpu.VMEM((KA, D), jnp.float32),
        pltpu.VMEM((KA, D), jnp.float32),
        pltpu.VMEM((KA, D), jnp.float32),
        pltpu.VMEM_SHARED((NP, D), jnp.float32),
        pltpu.SemaphoreType.DMA,
        pltpu.SemaphoreType.DMA,
        pltpu.SemaphoreType.DMA,
        pltpu.SemaphoreType.DMA,
    ],
)


def _dinv_of(deg_ref):
    return lax.rsqrt(deg_ref[0, :NN, 0:1] + deg_ref[1, :NN, 0:1] + 1.0)


def _tc1_body(x_ref, w1_ref, deg_ref, g1_ref):
    dinv = _dinv_of(deg_ref)
    h = jnp.dot(x_ref[...], w1_ref[...], preferred_element_type=jnp.float32)
    g1_ref[...] = h * dinv


_tc1_call = pl.pallas_call(
    _tc1_body,
    out_shape=jax.ShapeDtypeStruct((NN, D), jnp.float32),
)


def _tc2_body(s_ref, g1_ref, deg_ref, b1_ref, ga_ref, be_ref, w2_ref, g2_ref):
    dinv = _dinv_of(deg_ref)
    a = dinv * (s_ref[0, :NN, :] + s_ref[1, :NN, :] + g1_ref[...]) + b1_ref[...]
    mu = jnp.mean(a, axis=0, keepdims=True)
    var = jnp.mean((a - mu) ** 2, axis=0, keepdims=True)
    h = (a - mu) * lax.rsqrt(var + 1e-5) * ga_ref[...] + be_ref[...]
    h = jnp.maximum(h, 0.0)
    h2 = jnp.dot(h, w2_ref[...], preferred_element_type=jnp.float32)
    g2_ref[...] = h2 * dinv


_tc2_call = pl.pallas_call(
    _tc2_body,
    out_shape=jax.ShapeDtypeStruct((NN, D), jnp.float32),
)


def _tc3_body(s_ref, g2_ref, deg_ref, b2_ref, o_ref):
    dinv = _dinv_of(deg_ref)
    o_ref[...] = dinv * (s_ref[0, :NN, :] + s_ref[1, :NN, :] + g2_ref[...]) \
        + b2_ref[...]


_tc3_call = pl.pallas_call(
    _tc3_body,
    out_shape=jax.ShapeDtypeStruct((NN, D), jnp.float32),
)


def kernel(x, edge_index, W1, b1, gamma1, beta1, W2, b2):
    x2 = x.reshape(NN, D)
    src = edge_index[0]
    dst = edge_index[1]
    pad = EP - EE
    srcf = jnp.concatenate([src, jnp.zeros((pad,), jnp.int32)])
    dstf = jnp.concatenate([dst, jnp.full((pad,), NN, jnp.int32)])
    srcp = srcf.reshape(NW, CHA, KA)
    dstp = dstf.reshape(NW, CHA, KA)
    dstp_deg = dstf.reshape(NW, CH, K)
    z128 = jnp.zeros((NP, D), jnp.float32)
    onew = jnp.ones((K, D), jnp.float32)

    degA = _deg_call(dstp_deg, z128, onew)
    g1 = _tc1_call(x2, W1, degA)
    S1 = _agg_call(g1, srcp, dstp, z128)
    g2 = _tc2_call(S1, g1, degA, b1.reshape(1, D), gamma1.reshape(1, D),
                   beta1.reshape(1, D), W2)
    S2 = _agg_call(g2, srcp, dstp, z128)
    out = _tc3_call(S2, g2, degA, b2.reshape(1, D))
    return out.reshape(1, NN, D)


# speedup vs baseline: 9.8081x; 1.0244x over previous
import functools

import jax
import jax.numpy as jnp
from jax import lax
from jax.experimental import pallas as pl
from jax.experimental.pallas import tpu as pltpu
from jax.experimental.pallas import tpu_sc as plsc

NN = 10000
EE = 320000
D = 128
NC = 2
NS = 16
NW = NC * NS
K = 128
CH = 80
EP = NW * CH * K
NP = 10112
RPT = NP // NS

_MESH = plsc.VectorSubcoreMesh(core_axis_name="c", subcore_axis_name="s",
                               num_cores=NC, num_subcores=NS)


def _deg_body(dst_hbm, z_hbm, onew_hbm, deg_out, dst_v, one_v, degS, sem):
    c = lax.axis_index("c")
    s = lax.axis_index("s")
    wid = c * NS + s
    pltpu.sync_copy(dst_hbm.at[wid], dst_v)
    pltpu.sync_copy(onew_hbm, one_v)
    pltpu.sync_copy(z_hbm.at[pl.ds(s * RPT, RPT)], degS.at[pl.ds(s * RPT, RPT)])
    plsc.subcore_barrier()

    def body(j, carry):
        pltpu.sync_copy(one_v, degS.at[dst_v.at[j]], add=True)
        return carry

    lax.fori_loop(0, CH, body, 0)
    plsc.subcore_barrier()
    pltpu.sync_copy(degS.at[pl.ds(s * RPT, RPT)],
                    deg_out.at[c, pl.ds(s * RPT, RPT)])


_deg_call = pl.kernel(
    _deg_body,
    out_type=jax.ShapeDtypeStruct((NC, NP, D), jnp.float32),
    mesh=_MESH,
    scratch_types=[
        pltpu.VMEM((CH, K), jnp.int32),
        pltpu.VMEM((K, D), jnp.float32),
        pltpu.VMEM_SHARED((NP, D), jnp.float32),
        pltpu.SemaphoreType.DMA,
    ],
)


KA = 64
NBUF = 4
TCHUNKS = EP // KA
FC = 1
FCHUNKS = 256
SCHUNKS = 64
FBASE = NS * FCHUNKS
CHQ = 64
NQF = FCHUNKS // CHQ


def _agg_body(g_hbm, src_hbm, dst_hbm, z_hbm, s_out, src_v, dst_v, rows0,
              rows1, rows2, rows3, S, sem0, sem1, sem2, sem3):
    c = lax.axis_index("c")
    s = lax.axis_index("s")
    rows = (rows0, rows1, rows2, rows3)
    sems = (sem0, sem1, sem2, sem3)
    fast = c == FC
    nq = jnp.where(fast, NQF, 1)
    base = jnp.where(fast, s * FCHUNKS, FBASE + s * SCHUNKS)
    pltpu.sync_copy(z_hbm.at[pl.ds(s * RPT, RPT)], S.at[pl.ds(s * RPT, RPT)])
    plsc.subcore_barrier()

    for q in range(NQF):

        @pl.when(q < nq)
        def _():
            pltpu.sync_copy(src_hbm.at[pl.ds(base + q * CHQ, CHQ)], src_v)
            pltpu.sync_copy(dst_hbm.at[pl.ds(base + q * CHQ, CHQ)], dst_v)
            for b in range(NBUF):
                pltpu.async_copy(g_hbm.at[src_v.at[b]], rows[b], sems[b])

            def body(t, carry):
                j0 = NBUF * t
                for b in range(NBUF):
                    j = j0 + b
                    pltpu.make_async_copy(g_hbm.at[src_v.at[j]], rows[b],
                                          sems[b]).wait()
                    pltpu.sync_copy(rows[b], S.at[dst_v.at[j]], add=True)

                    @pl.when(j + NBUF < CHQ)
                    def _():
                        pltpu.async_copy(g_hbm.at[src_v.at[j + NBUF]],
                                         rows[b], sems[b])

                return carry

            lax.fori_loop(0, CHQ // NBUF, body, 0)

    plsc.subcore_barrier()
    pltpu.sync_copy(S.at[pl.ds(s * RPT, RPT)],
                    s_out.at[c, pl.ds(s * RPT, RPT)])


_agg_call = pl.kernel(
    _agg_body,
    out_type=jax.ShapeDtypeStruct((NC, NP, D), jnp.float32),
    mesh=_MESH,
    scratch_types=[
        pltpu.VMEM((CHQ, KA), jnp.int32),
        pltpu.VMEM((CHQ, KA), jnp.int32),
        pltpu.VMEM((KA, D), jnp.float32),
        pltpu.VMEM((KA, D), jnp.float32),
        pltpu.VMEM((KA, D), jnp.float32),
        pltpu.VMEM((KA, D), jnp.float32),
        pltpu.VMEM_SHARED((NP, D), jnp.float32),
        pltpu.SemaphoreType.DMA,
        pltpu.SemaphoreType.DMA,
        pltpu.SemaphoreType.DMA,
        pltpu.SemaphoreType.DMA,
    ],
)


def _dinv_of(deg_ref):
    return lax.rsqrt(deg_ref[0, :NN, 0:1] + deg_ref[1, :NN, 0:1] + 1.0)


def _tc1_body(x_ref, w1_ref, deg_ref, g1_ref):
    dinv = _dinv_of(deg_ref)
    h = jnp.dot(x_ref[...], w1_ref[...], preferred_element_type=jnp.float32)
    g1_ref[...] = h * dinv


_tc1_call = pl.pallas_call(
    _tc1_body,
    out_shape=jax.ShapeDtypeStruct((NN, D), jnp.float32),
)


def _tc2_body(s_ref, g1_ref, deg_ref, b1_ref, ga_ref, be_ref, w2_ref, g2_ref):
    dinv = _dinv_of(deg_ref)
    a = dinv * (s_ref[0, :NN, :] + s_ref[1, :NN, :] + g1_ref[...]) + b1_ref[...]
    mu = jnp.mean(a, axis=0, keepdims=True)
    var = jnp.mean((a - mu) ** 2, axis=0, keepdims=True)
    h = (a - mu) * lax.rsqrt(var + 1e-5) * ga_ref[...] + be_ref[...]
    h = jnp.maximum(h, 0.0)
    h2 = jnp.dot(h, w2_ref[...], preferred_element_type=jnp.float32)
    g2_ref[...] = h2 * dinv


_tc2_call = pl.pallas_call(
    _tc2_body,
    out_shape=jax.ShapeDtypeStruct((NN, D), jnp.float32),
)


def _tc3_body(s_ref, g2_ref, deg_ref, b2_ref, o_ref):
    dinv = _dinv_of(deg_ref)
    o_ref[...] = dinv * (s_ref[0, :NN, :] + s_ref[1, :NN, :] + g2_ref[...]) \
        + b2_ref[...]


_tc3_call = pl.pallas_call(
    _tc3_body,
    out_shape=jax.ShapeDtypeStruct((NN, D), jnp.float32),
)


def kernel(x, edge_index, W1, b1, gamma1, beta1, W2, b2):
    x2 = x.reshape(NN, D)
    src = edge_index[0]
    dst = edge_index[1]
    pad = EP - EE
    srcf = jnp.concatenate([src, jnp.zeros((pad,), jnp.int32)])
    trash = NN + jnp.arange(pad, dtype=jnp.int32) % (NP - NN)
    dstf = jnp.concatenate([dst, trash])
    srcp = srcf.reshape(TCHUNKS, KA)
    dstp = dstf.reshape(TCHUNKS, KA)
    dstp_deg = dstf.reshape(NW, CH, K)
    z128 = jnp.zeros((NP, D), jnp.float32)
    onew = jnp.ones((K, D), jnp.float32)

    degA = _deg_call(dstp_deg, z128, onew)
    g1 = _tc1_call(x2, W1, degA)
    S1 = _agg_call(g1, srcp, dstp, z128)
    g2 = _tc2_call(S1, g1, degA, b1.reshape(1, D), gamma1.reshape(1, D),
                   beta1.reshape(1, D), W2)
    S2 = _agg_call(g2, srcp, dstp, z128)
    out = _tc3_call(S2, g2, degA, b2.reshape(1, D))
    return out.reshape(1, NN, D)


# speedup vs baseline: 10.6375x; 1.0846x over previous
import functools

import jax
import jax.numpy as jnp
from jax import lax
from jax.experimental import pallas as pl
from jax.experimental.pallas import tpu as pltpu
from jax.experimental.pallas import tpu_sc as plsc

NN = 10000
EE = 320000
D = 128
NC = 2
NS = 16
NW = NC * NS
K = 128
CH = 80
EP = NW * CH * K
NP = 10112
RPT = NP // NS

_MESH = plsc.VectorSubcoreMesh(core_axis_name="c", subcore_axis_name="s",
                               num_cores=NC, num_subcores=NS)


def _deg_body(dst_hbm, z_hbm, onew_hbm, deg_out, dst_v, one_v, degS, sem):
    c = lax.axis_index("c")
    s = lax.axis_index("s")
    wid = c * NS + s
    pltpu.sync_copy(dst_hbm.at[wid], dst_v)
    pltpu.sync_copy(onew_hbm, one_v)
    pltpu.sync_copy(z_hbm.at[pl.ds(s * RPT, RPT)], degS.at[pl.ds(s * RPT, RPT)])
    plsc.subcore_barrier()

    def body(j, carry):
        pltpu.sync_copy(one_v, degS.at[dst_v.at[j]], add=True)
        return carry

    lax.fori_loop(0, CH, body, 0)
    plsc.subcore_barrier()
    pltpu.sync_copy(degS.at[pl.ds(s * RPT, RPT)],
                    deg_out.at[c, pl.ds(s * RPT, RPT)])


_deg_call = pl.kernel(
    _deg_body,
    out_type=jax.ShapeDtypeStruct((NC, NP, D), jnp.float32),
    mesh=_MESH,
    scratch_types=[
        pltpu.VMEM((CH, K), jnp.int32),
        pltpu.VMEM((K, D), jnp.float32),
        pltpu.VMEM_SHARED((NP, D), jnp.float32),
        pltpu.SemaphoreType.DMA,
    ],
)


KA = 64
NBUF = 4
TCHUNKS = EP // KA
FC = 1
FCHUNKS = 256
SCHUNKS = 64
FBASE = NS * FCHUNKS
CHQ = 64
NQF = FCHUNKS // CHQ


def _agg_body(g_hbm, src_hbm, dst_hbm, z_hbm, s_out, src_v, dst_v, rows0,
              rows1, rows2, rows3, S, sem0, sem1, sem2, sem3):
    c = lax.axis_index("c")
    s = lax.axis_index("s")
    rows = (rows0, rows1, rows2, rows3)
    sems = (sem0, sem1, sem2, sem3)
    fast = c == FC
    nq = jnp.where(fast, NQF, 1)
    base = jnp.where(fast, s * FCHUNKS, FBASE + s * SCHUNKS)
    pltpu.sync_copy(z_hbm.at[pl.ds(s * RPT, RPT)], S.at[pl.ds(s * RPT, RPT)])
    plsc.subcore_barrier()

    for q in range(NQF):

        @pl.when(q < nq)
        def _():
            pltpu.sync_copy(src_hbm.at[pl.ds(base + q * CHQ, CHQ)], src_v)
            pltpu.sync_copy(dst_hbm.at[pl.ds(base + q * CHQ, CHQ)], dst_v)
            for b in range(NBUF):
                pltpu.async_copy(g_hbm.at[src_v.at[b]], rows[b], sems[b])

            def body(t, carry):
                j0 = NBUF * t
                for b in range(NBUF):
                    j = j0 + b
                    pltpu.make_async_copy(g_hbm.at[src_v.at[j]], rows[b],
                                          sems[b]).wait()
                    pltpu.sync_copy(rows[b], S.at[dst_v.at[j]], add=True)

                    @pl.when(j + NBUF < CHQ)
                    def _():
                        pltpu.async_copy(g_hbm.at[src_v.at[j + NBUF]],
                                         rows[b], sems[b])

                return carry

            lax.fori_loop(0, CHQ // NBUF, body, 0)

    plsc.subcore_barrier()
    pltpu.sync_copy(S.at[pl.ds(s * RPT, RPT)],
                    s_out.at[c, pl.ds(s * RPT, RPT)])


_agg_call = pl.kernel(
    _agg_body,
    out_type=jax.ShapeDtypeStruct((NC, NP, D), jnp.float32),
    mesh=_MESH,
    scratch_types=[
        pltpu.VMEM((CHQ, KA), jnp.int32),
        pltpu.VMEM((CHQ, KA), jnp.int32),
        pltpu.VMEM((KA, D), jnp.float32),
        pltpu.VMEM((KA, D), jnp.float32),
        pltpu.VMEM((KA, D), jnp.float32),
        pltpu.VMEM((KA, D), jnp.float32),
        pltpu.VMEM_SHARED((NP, D), jnp.float32),
        pltpu.SemaphoreType.DMA,
        pltpu.SemaphoreType.DMA,
        pltpu.SemaphoreType.DMA,
        pltpu.SemaphoreType.DMA,
    ],
)


def _dinv_of(deg_ref):
    return lax.rsqrt(deg_ref[0, :NN, 0:1] + deg_ref[1, :NN, 0:1] + 1.0)


def _tcmm_body(x_ref, w1_ref, h_ref):
    h_ref[...] = jnp.dot(x_ref[...], w1_ref[...],
                         preferred_element_type=jnp.float32)


_tcmm_call = pl.pallas_call(
    _tcmm_body,
    out_shape=jax.ShapeDtypeStruct((NN, D), jnp.float32),
)


def _tcg_body(h_ref, deg_ref, g1_ref):
    g1_ref[...] = h_ref[...] * _dinv_of(deg_ref)


_tcg_call = pl.pallas_call(
    _tcg_body,
    out_shape=jax.ShapeDtypeStruct((NN, D), jnp.float32),
)


def _tc2_body(s_ref, g1_ref, deg_ref, b1_ref, ga_ref, be_ref, w2_ref, g2_ref):
    dinv = _dinv_of(deg_ref)
    a = dinv * (s_ref[0, :NN, :] + s_ref[1, :NN, :] + g1_ref[...]) + b1_ref[...]
    mu = jnp.mean(a, axis=0, keepdims=True)
    var = jnp.mean((a - mu) ** 2, axis=0, keepdims=True)
    h = (a - mu) * lax.rsqrt(var + 1e-5) * ga_ref[...] + be_ref[...]
    h = jnp.maximum(h, 0.0)
    h2 = jnp.dot(h, w2_ref[...], preferred_element_type=jnp.float32)
    g2_ref[...] = h2 * dinv


_tc2_call = pl.pallas_call(
    _tc2_body,
    out_shape=jax.ShapeDtypeStruct((NN, D), jnp.float32),
)


def _tc3_body(s_ref, g2_ref, deg_ref, b2_ref, o_ref):
    dinv = _dinv_of(deg_ref)
    o_ref[...] = dinv * (s_ref[0, :NN, :] + s_ref[1, :NN, :] + g2_ref[...]) \
        + b2_ref[...]


_tc3_call = pl.pallas_call(
    _tc3_body,
    out_shape=jax.ShapeDtypeStruct((NN, D), jnp.float32),
)


def kernel(x, edge_index, W1, b1, gamma1, beta1, W2, b2):
    x2 = x.reshape(NN, D)
    src = edge_index[0]
    dst = edge_index[1]
    pad = EP - EE
    srcf = jnp.concatenate([src, jnp.zeros((pad,), jnp.int32)])
    trash = NN + jnp.arange(pad, dtype=jnp.int32) % (NP - NN)
    dstf = jnp.concatenate([dst, trash])
    srcp = srcf.reshape(TCHUNKS, KA)
    dstp = dstf.reshape(TCHUNKS, KA)
    dstp_deg = dstf.reshape(NW, CH, K)
    z128 = jnp.zeros((NP, D), jnp.float32)
    onew = jnp.ones((K, D), jnp.float32)

    degA = _deg_call(dstp_deg, z128, onew)
    h1 = _tcmm_call(x2, W1)
    g1 = _tcg_call(h1, degA)
    S1 = _agg_call(g1, srcp, dstp, z128)
    g2 = _tc2_call(S1, g1, degA, b1.reshape(1, D), gamma1.reshape(1, D),
                   beta1.reshape(1, D), W2)
    S2 = _agg_call(g2, srcp, dstp, z128)
    out = _tc3_call(S2, g2, degA, b2.reshape(1, D))
    return out.reshape(1, NN, D)


# speedup vs baseline: 11.1460x; 1.0478x over previous
import functools

import jax
import jax.numpy as jnp
from jax import lax
from jax.experimental import pallas as pl
from jax.experimental.pallas import tpu as pltpu
from jax.experimental.pallas import tpu_sc as plsc

NN = 10000
EE = 320000
D = 128
NC = 2
NS = 16
NW = NC * NS
K = 128
CH = 80
EP = NW * CH * K
NP = 10112
RPT = NP // NS

_MESH = plsc.VectorSubcoreMesh(core_axis_name="c", subcore_axis_name="s",
                               num_cores=NC, num_subcores=NS)


def _deg_body(dst_hbm, z_hbm, onew_hbm, deg_out, dst_v, one_v, degS, sem):
    c = lax.axis_index("c")
    s = lax.axis_index("s")
    wid = c * NS + s
    pltpu.sync_copy(dst_hbm.at[wid], dst_v)
    pltpu.sync_copy(onew_hbm, one_v)
    pltpu.sync_copy(z_hbm.at[pl.ds(s * RPT, RPT)], degS.at[pl.ds(s * RPT, RPT)])
    plsc.subcore_barrier()

    def body(j, carry):
        pltpu.sync_copy(one_v, degS.at[dst_v.at[j]], add=True)
        return carry

    lax.fori_loop(0, CH, body, 0)
    plsc.subcore_barrier()
    pltpu.sync_copy(degS.at[pl.ds(s * RPT, RPT)],
                    deg_out.at[c, pl.ds(s * RPT, RPT)])


_deg_call = pl.kernel(
    _deg_body,
    out_type=jax.ShapeDtypeStruct((NC, NP, D), jnp.float32),
    mesh=_MESH,
    scratch_types=[
        pltpu.VMEM((CH, K), jnp.int32),
        pltpu.VMEM((K, D), jnp.float32),
        pltpu.VMEM_SHARED((NP, D), jnp.float32),
        pltpu.SemaphoreType.DMA,
    ],
)


KA = 64
NBUF = 4
TCHUNKS = EP // KA
FC = 0
FCHUNKS = 256
SCHUNKS = 64
FBASE = NS * FCHUNKS
CHQ = 64
NQF = FCHUNKS // CHQ


def _agg_body(g_hbm, src_hbm, dst_hbm, z_hbm, s_out, src_v, dst_v, rows0,
              rows1, rows2, rows3, S, sem0, sem1, sem2, sem3):
    c = lax.axis_index("c")
    s = lax.axis_index("s")
    rows = (rows0, rows1, rows2, rows3)
    sems = (sem0, sem1, sem2, sem3)
    fast = c == FC
    nq = jnp.where(fast, NQF, 1)
    base = jnp.where(fast, s * FCHUNKS, FBASE + s * SCHUNKS)
    pltpu.sync_copy(z_hbm.at[pl.ds(s * RPT, RPT)], S.at[pl.ds(s * RPT, RPT)])
    plsc.subcore_barrier()

    for q in range(NQF):

        @pl.when(q < nq)
        def _():
            pltpu.sync_copy(src_hbm.at[pl.ds(base + q * CHQ, CHQ)], src_v)
            pltpu.sync_copy(dst_hbm.at[pl.ds(base + q * CHQ, CHQ)], dst_v)
            for b in range(NBUF):
                pltpu.async_copy(g_hbm.at[src_v.at[b]], rows[b], sems[b])

            def body(t, carry):
                j0 = NBUF * t
                for b in range(NBUF):
                    j = j0 + b
                    pltpu.make_async_copy(g_hbm.at[src_v.at[j]], rows[b],
                                          sems[b]).wait()
                    pltpu.sync_copy(rows[b], S.at[dst_v.at[j]], add=True)

                    @pl.when(j + NBUF < CHQ)
                    def _():
                        pltpu.async_copy(g_hbm.at[src_v.at[j + NBUF]],
                                         rows[b], sems[b])

                return carry

            lax.fori_loop(0, CHQ // NBUF, body, 0)

    plsc.subcore_barrier()
    pltpu.sync_copy(S.at[pl.ds(s * RPT, RPT)],
                    s_out.at[c, pl.ds(s * RPT, RPT)])


_agg_call = pl.kernel(
    _agg_body,
    out_type=jax.ShapeDtypeStruct((NC, NP, D), jnp.float32),
    mesh=_MESH,
    scratch_types=[
        pltpu.VMEM((CHQ, KA), jnp.int32),
        pltpu.VMEM((CHQ, KA), jnp.int32),
        pltpu.VMEM((KA, D), jnp.float32),
        pltpu.VMEM((KA, D), jnp.float32),
        pltpu.VMEM((KA, D), jnp.float32),
        pltpu.VMEM((KA, D), jnp.float32),
        pltpu.VMEM_SHARED((NP, D), jnp.float32),
        pltpu.SemaphoreType.DMA,
        pltpu.SemaphoreType.DMA,
        pltpu.SemaphoreType.DMA,
        pltpu.SemaphoreType.DMA,
    ],
)


def _dinv_of(deg_ref):
    return lax.rsqrt(deg_ref[0, :NN, 0:1] + deg_ref[1, :NN, 0:1] + 1.0)


def _tcmm_body(x_ref, w1_ref, h_ref):
    h_ref[...] = jnp.dot(x_ref[...], w1_ref[...],
                         preferred_element_type=jnp.float32)


_tcmm_call = pl.pallas_call(
    _tcmm_body,
    out_shape=jax.ShapeDtypeStruct((NN, D), jnp.float32),
)


def _tcg_body(h_ref, deg_ref, g1_ref):
    g1_ref[...] = h_ref[...] * _dinv_of(deg_ref)


_tcg_call = pl.pallas_call(
    _tcg_body,
    out_shape=jax.ShapeDtypeStruct((NN, D), jnp.float32),
)


def _tc2_body(s_ref, g1_ref, deg_ref, b1_ref, ga_ref, be_ref, w2_ref, g2_ref):
    dinv = _dinv_of(deg_ref)
    a = dinv * (s_ref[0, :NN, :] + s_ref[1, :NN, :] + g1_ref[...]) + b1_ref[...]
    mu = jnp.mean(a, axis=0, keepdims=True)
    var = jnp.mean((a - mu) ** 2, axis=0, keepdims=True)
    h = (a - mu) * lax.rsqrt(var + 1e-5) * ga_ref[...] + be_ref[...]
    h = jnp.maximum(h, 0.0)
    h2 = jnp.dot(h, w2_ref[...], preferred_element_type=jnp.float32)
    g2_ref[...] = h2 * dinv


_tc2_call = pl.pallas_call(
    _tc2_body,
    out_shape=jax.ShapeDtypeStruct((NN, D), jnp.float32),
)


def _tc3_body(s_ref, g2_ref, deg_ref, b2_ref, o_ref):
    dinv = _dinv_of(deg_ref)
    o_ref[...] = dinv * (s_ref[0, :NN, :] + s_ref[1, :NN, :] + g2_ref[...]) \
        + b2_ref[...]


_tc3_call = pl.pallas_call(
    _tc3_body,
    out_shape=jax.ShapeDtypeStruct((NN, D), jnp.float32),
)


def kernel(x, edge_index, W1, b1, gamma1, beta1, W2, b2):
    x2 = x.reshape(NN, D)
    src = edge_index[0]
    dst = edge_index[1]
    pad = EP - EE
    srcf = jnp.concatenate([src, jnp.zeros((pad,), jnp.int32)])
    trash = NN + jnp.arange(pad, dtype=jnp.int32) % (NP - NN)
    dstf = jnp.concatenate([dst, trash])
    srcp = srcf.reshape(TCHUNKS, KA)
    dstp = dstf.reshape(TCHUNKS, KA)
    dstp_deg = dstf.reshape(NW, CH, K)
    z128 = jnp.zeros((NP, D), jnp.float32)
    onew = jnp.ones((K, D), jnp.float32)

    degA = _deg_call(dstp_deg, z128, onew)
    h1 = _tcmm_call(x2, W1)
    g1 = _tcg_call(h1, degA)
    S1 = _agg_call(g1, srcp, dstp, z128)
    g2 = _tc2_call(S1, g1, degA, b1.reshape(1, D), gamma1.reshape(1, D),
                   beta1.reshape(1, D), W2)
    S2 = _agg_call(g2, srcp, dstp, z128)
    out = _tc3_call(S2, g2, degA, b2.reshape(1, D))
    return out.reshape(1, NN, D)
